# baseline (speedup 1.0000x reference)
import functools

import jax
import jax.numpy as jnp
from jax import lax
from jax.experimental import pallas as pl
from jax.experimental.pallas import tpu as pltpu
from jax.experimental.pallas import tpu_sc as plsc

N = 10000
E = 320000
D = 128
H = 64
NC = 2
NS = 16
CHUNK = 32
E_PER_SUB = E // NS
SUPE = 800
SUPC = SUPE // CHUNK
NSUPS = E_PER_SUB // SUPE
NCH = E_PER_SUB // CHUNK
NPAIR = (NCH - 1) // 2
STRIPE = 624
STRIPE_LAST = N - 15 * STRIPE
F32 = jnp.float32


def _emb_body(nf_ref, wh_ref, bh_ref, we_ref, cw_ref, cb_ref, h_ref, uv_ref):
    x = nf_ref[...]
    h_ref[...] = jnp.dot(x, wh_ref[...].T, preferred_element_type=F32) + bh_ref[...]
    we = we_ref[...]
    cw = cw_ref[...]
    u = jnp.dot(we[0:1], cw.T, preferred_element_type=F32)[0]
    v = jnp.dot(we[1:2], cw.T, preferred_element_type=F32)[0] + cb_ref[0]
    uv_ref[...] = jnp.stack([u[:H], u[H:], v[:H], v[H:]], axis=0)


def _emb_call(nodes_feat, wh, bh, we2, cw, cb):
    blk = 2000
    grid = N // blk
    return pl.pallas_call(
        _emb_body,
        grid=(grid,),
        in_specs=[
            pl.BlockSpec((blk, D), lambda i: (i, 0)),
            pl.BlockSpec((D, D), lambda i: (0, 0)),
            pl.BlockSpec((1, D), lambda i: (0, 0)),
            pl.BlockSpec((2, D), lambda i: (0, 0)),
            pl.BlockSpec((D, D), lambda i: (0, 0)),
            pl.BlockSpec((1, D), lambda i: (0, 0)),
        ],
        out_specs=[
            pl.BlockSpec((blk, D), lambda i: (i, 0)),
            pl.BlockSpec((4, H), lambda i: (0, 0)),
        ],
        out_shape=[
            jax.ShapeDtypeStruct((N, D), F32),
            jax.ShapeDtypeStruct((4, H), F32),
        ],
    )(nodes_feat, wh, bh, we2, cw, cb)


def _proj_body(h_ref, w_ref, b_ref, a_ref, db_ref, es_ref):
    x = h_ref[...]
    w = w_ref[...]
    b = b_ref[...]
    a_ref[...] = jnp.dot(x, w[0:D].T, preferred_element_type=F32) + b[0]
    bh = jnp.dot(x, w[D:2 * D].T, preferred_element_type=F32) + b[1]
    dh = jnp.dot(x, w[2 * D:3 * D].T, preferred_element_type=F32) + b[2]
    eh = jnp.dot(x, w[3 * D:4 * D].T, preferred_element_type=F32) + b[3]
    db_ref[...] = jnp.stack(
        [jnp.concatenate([dh[:, :H], bh[:, :H]], axis=1),
         jnp.concatenate([dh[:, H:], bh[:, H:]], axis=1)], axis=0)
    es_ref[...] = jnp.stack([eh[:, :H], eh[:, H:]], axis=0)


def _proj_call(h, w4, b4):
    blk = 2000
    grid = N // blk
    return pl.pallas_call(
        _proj_body,
        grid=(grid,),
        in_specs=[
            pl.BlockSpec((blk, D), lambda i: (i, 0)),
            pl.BlockSpec((4 * D, D), lambda i: (0, 0)),
            pl.BlockSpec((4, D), lambda i: (0, 0)),
        ],
        out_specs=[
            pl.BlockSpec((blk, D), lambda i: (i, 0)),
            pl.BlockSpec((2, blk, D), lambda i: (0, i, 0)),
            pl.BlockSpec((2, blk, H), lambda i: (0, i, 0)),
        ],
        out_shape=[
            jax.ShapeDtypeStruct((N, D), F32),
            jax.ShapeDtypeStruct((2, N, D), F32),
            jax.ShapeDtypeStruct((2, N, H), F32),
        ],
    )(h, w4, b4)


def _estats_body(eij_ref, en_ref, st_ref):
    i = pl.program_id(0)

    @pl.when(i == 0)
    def _():
        st_ref[...] = jnp.zeros_like(st_ref)

    eij = eij_ref[...]
    x = jnp.concatenate([eij[0], eij[1]], axis=1) * en_ref[...]
    st_ref[0:1, :] += jnp.sum(x, axis=0, keepdims=True)
    st_ref[1:2, :] += jnp.sum(jnp.square(x), axis=0, keepdims=True)


def _estats_call(eij, en):
    blk = 4000
    grid = E // blk
    return pl.pallas_call(
        _estats_body,
        grid=(grid,),
        in_specs=[
            pl.BlockSpec((2, blk, H), lambda i: (0, i, 0)),
            pl.BlockSpec((blk, 1), lambda i: (i, 0)),
        ],
        out_specs=pl.BlockSpec((8, D), lambda i: (0, 0)),
        out_shape=jax.ShapeDtypeStruct((8, D), F32),
    )(eij, en)


def _hupd_body(acc_ref, ah_ref, hin_ref, nn_ref, gb_ref, out_ref):
    acc = acc_ref[...]
    num = jnp.concatenate([acc[0, :, :H], acc[1, :, :H]], axis=1)
    den = jnp.concatenate([acc[0, :, H:], acc[1, :, H:]], axis=1)
    x = (ah_ref[...] + num / (den + 1e-6)) * nn_ref[...]
    mu = jnp.mean(x, axis=0, keepdims=True)
    var = jnp.mean(jnp.square(x), axis=0, keepdims=True) - jnp.square(mu)
    gb = gb_ref[...]
    x = (x - mu) / jnp.sqrt(var + 1e-5) * gb[0:1] + gb[1:2]
    out_ref[...] = hin_ref[...] + jnp.maximum(x, 0.0)


def _hupdate(acc, ah, h_in, nn, gb):
    return pl.pallas_call(
        _hupd_body,
        out_shape=jax.ShapeDtypeStruct((N, D), F32),
    )(acc, ah, h_in, nn, gb)


def _final_body(acc_ref, ah_ref, hin_ref, nn_ref, gb_ref, ow_ref, out_ref):
    acc = acc_ref[...]
    num = jnp.concatenate([acc[0, :, :H], acc[1, :, :H]], axis=1)
    den = jnp.concatenate([acc[0, :, H:], acc[1, :, H:]], axis=1)
    x = (ah_ref[...] + num / (den + 1e-6)) * nn_ref[...]
    mu = jnp.mean(x, axis=0, keepdims=True)
    var = jnp.mean(jnp.square(x), axis=0, keepdims=True) - jnp.square(mu)
    gb = gb_ref[...]
    x = (x - mu) / jnp.sqrt(var + 1e-5) * gb[0:1] + gb[1:2]
    h = hin_ref[...] + jnp.maximum(x, 0.0)
    hg = jnp.mean(h, axis=0, keepdims=True)
    out_ref[...] = jnp.dot(hg, ow_ref[...].T, preferred_element_type=F32)


def _final_call(acc, ah, h_in, nn, gb, out_w):
    n_classes = out_w.shape[0]
    return pl.pallas_call(
        _final_body,
        out_shape=jax.ShapeDtypeStruct((1, n_classes), F32),
    )(acc, ah, h_in, nn, gb, out_w)


def _ce1_body(eij_ref, ef_ref, en_ref, st_ref, sc_ref, cw_ref, ce_ref):
    st = st_ref[...]
    mean = st[0] / float(E)
    var = st[1] / float(E) - jnp.square(mean)
    inv = 1.0 / jnp.sqrt(var + 1e-5)

    sc = sc_ref[...]
    eij = eij_ref[...]
    x = jnp.concatenate([eij[0], eij[1]], axis=1) * en_ref[...]
    x = (x - mean[None, :]) * inv[None, :] * sc[2:3] + sc[3:4]
    e0 = ef_ref[...] * sc[0:1] + sc[1:2]
    el1 = e0 + jnp.maximum(x, 0.0)
    ce = jnp.dot(el1, cw_ref[...].T, preferred_element_type=F32) + sc[4:5]
    ce_ref[...] = jnp.stack([ce[:, :H], ce[:, H:]], axis=0)


def _ce1_call(eij, ef, en, stats, scal, cw):
    blk = 2000
    grid = E // blk
    return pl.pallas_call(
        _ce1_body,
        grid=(grid,),
        in_specs=[
            pl.BlockSpec((2, blk, H), lambda i: (0, i, 0)),
            pl.BlockSpec((blk, 1), lambda i: (i, 0)),
            pl.BlockSpec((blk, 1), lambda i: (i, 0)),
            pl.BlockSpec((8, D), lambda i: (0, 0)),
            pl.BlockSpec((5, D), lambda i: (0, 0)),
            pl.BlockSpec((D, D), lambda i: (0, 0)),
        ],
        out_specs=pl.BlockSpec((2, blk, H), lambda i: (0, i, 0)),
        out_shape=jax.ShapeDtypeStruct((2, E, H), F32),
    )(eij, ef, en, stats, scal, cw)


def _sigmoid(x):
    return 1.0 / (1.0 + jnp.exp(-x))


def _sc_edge(idxp, ce_or_uv, tdb, te, with_eij, efp=None):
    mesh = plsc.VectorSubcoreMesh(
        core_axis_name="c", subcore_axis_name="s", num_cores=NC,
        num_subcores=NS)

    out_type = [jax.ShapeDtypeStruct((NC * N, D), F32)]
    if with_eij:
        out_type.append(jax.ShapeDtypeStruct((NC * E, H), F32))

    scratch = [
        pltpu.VMEM((2, SUPE), jnp.int32),
        pltpu.VMEM((CHUNK,), jnp.int32),
        pltpu.VMEM((CHUNK,), jnp.int32),
        pltpu.VMEM((CHUNK,), jnp.int32),
        pltpu.VMEM((CHUNK,), jnp.int32),
        pltpu.VMEM((CHUNK,), jnp.int32),
        pltpu.VMEM((CHUNK,), jnp.int32),
        pltpu.VMEM((CHUNK, D), F32),
        pltpu.VMEM((CHUNK, D), F32),
        pltpu.VMEM((CHUNK, H), F32),
        pltpu.VMEM((CHUNK, H), F32),
        pltpu.VMEM((CHUNK, H), F32),
        pltpu.VMEM((CHUNK, H), F32),
        pltpu.VMEM((CHUNK, D), F32),
        pltpu.VMEM((CHUNK, D), F32),
        pltpu.VMEM((CHUNK,), jnp.int32),
        pltpu.VMEM((CHUNK,), jnp.int32),
        pltpu.VMEM_SHARED((N, D), F32),
    ]
    if with_eij:
        scratch += [
            pltpu.VMEM((CHUNK, H), F32),
            pltpu.VMEM((CHUNK, H), F32),
            pltpu.VMEM((H,), F32),
            pltpu.VMEM((H,), F32),
            pltpu.VMEM((SUPE,), F32),
        ]
    nsem = 8 if with_eij else 8
    scratch += [pltpu.SemaphoreType.DMA] * nsem

    def _body(idxp_h, ce_h, uv_h, efp_h, tdb_h, te_h, acc_h, eij_h,
              supidx, SV, DE, DV, DB, EH, CV, CT, SIX, acc_s, EO,
              uvec, vvec, supef, GS, ES, CS, WS):
        c = lax.axis_index("c")
        s = lax.axis_index("s")
        cN = c * N

        if with_eij:
            pltpu.sync_copy(uv_h.at[c], uvec)
            pltpu.sync_copy(uv_h.at[2 + c], vvec)

        ct0 = CT[0]

        def zrow(i, _):
            for f in range(D // 16):
                ct0[i, pl.ds(f * 16, 16)] = jnp.zeros((16,), F32)
            return 0
        lax.fori_loop(0, 16, zrow, 0)

        nz = lax.select(s == NS - 1, STRIPE_LAST // 16, STRIPE // 16)

        def zcopy(kk, _):
            pltpu.sync_copy(ct0.at[pl.ds(0, 16)],
                            acc_s.at[pl.ds(s * STRIPE + kk * 16, 16)])
            return 0
        lax.fori_loop(0, nz, zcopy, 0)

        plsc.subcore_barrier()

        def build_and_issue(ip1, q):
            off1 = (ip1 % SUPC) * CHUNK
            for g in range(CHUNK // 16):
                sl = pl.ds(off1 + g * 16, 16)
                dsl = pl.ds(g * 16, 16)
                srcs = supidx[0, sl]
                dsts = supidx[1, sl]
                SV[q][dsl] = srcs + cN
                DV[q][dsl] = dsts
                DE[q][dsl] = dsts + cN
            pltpu.async_copy(tdb_h.at[SV[q]], DB[q], GS[q][0])
            pltpu.async_copy(te_h.at[DE[q]], EH[q], GS[q][1])
            if not with_eij:
                cbase1 = c * E + s * E_PER_SUB + ip1 * CHUNK
                pltpu.async_copy(ce_h.at[pl.ds(cbase1, CHUNK)], CV[q], ES[q])

        def wait_in(p):
            pltpu.make_async_copy(
                tdb_h.at[pl.ds(0, CHUNK)], DB[p], GS[p][0]).wait()
            pltpu.make_async_copy(
                te_h.at[pl.ds(0, CHUNK)], EH[p], GS[p][1]).wait()
            if not with_eij:
                pltpu.make_async_copy(
                    ce_h.at[pl.ds(0, CHUNK)], CV[p], ES[p]).wait()

        def compute(i, p):
            db, eh, cv, ct = DB[p], EH[p], CV[p], CT[p]
            eo = EO[p] if with_eij else None
            off = (i % SUPC) * CHUNK

            if with_eij:
                def grp(g, carry):
                    ef16 = supef[pl.ds(off + g * 16, 16)]
                    for j in range(16):
                        efs = ef16[j]
                        r = g * 16 + j
                        for f in range(H // 16):
                            sl = pl.ds(f * 16, 16)
                            sl2 = pl.ds(H + f * 16, 16)
                            ce = efs * uvec[sl] + vvec[sl]
                            eij = db[r, sl] + eh[r, sl] + ce
                            sig = _sigmoid(eij)
                            eo[r, sl] = eij
                            ct[r, sl] = sig * db[r, sl2]
                            ct[r, sl2] = sig
                    return carry

                lax.fori_loop(0, CHUNK // 16, grp, 0)
            else:
                def grp(g, carry):
                    for j in range(4):
                        r = g * 4 + j
                        for f in range(H // 16):
                            sl = pl.ds(f * 16, 16)
                            sl2 = pl.ds(H + f * 16, 16)
                            eij = db[r, sl] + eh[r, sl] + cv[r, sl]
                            sig = _sigmoid(eij)
                            ct[r, sl] = sig * db[r, sl2]
                            ct[r, sl2] = sig
                    return carry

                lax.fori_loop(0, CHUNK // 4, grp, 0)

        def issue_writes(i, p):
            for g in range(CHUNK // 16):
                dsl = pl.ds(g * 16, 16)
                SIX[p][dsl] = DV[p][dsl]
            pltpu.async_copy(CT[p], acc_s.at[SIX[p]], CS[p], add=True)
            if with_eij:
                cbase = c * E + s * E_PER_SUB + i * CHUNK
                pltpu.async_copy(EO[p], eij_h.at[pl.ds(cbase, CHUNK)], WS[p])

        def wait_writes(p):
            pltpu.make_async_copy(
                CT[p], acc_s.at[pl.ds(0, CHUNK)], CS[p]).wait()
            if with_eij:
                pltpu.make_async_copy(
                    EO[p], eij_h.at[pl.ds(0, CHUNK)], WS[p]).wait()

        def subiter(i, p, issue_next):
            if issue_next:
                ip1 = i + 1

                @pl.when((ip1 % SUPC) == 0)
                def _():
                    pltpu.sync_copy(idxp_h.at[s * NSUPS + ip1 // SUPC], supidx)

                build_and_issue(ip1, p ^ 1)

            if with_eij:
                @pl.when((i % SUPC) == 0)
                def _():
                    pltpu.sync_copy(efp_h.at[s * NSUPS + i // SUPC], supef)

            wait_in(p)

            @pl.when(i >= 2)
            def _():
                wait_writes(p)

            compute(i, p)
            issue_writes(i, p)

        pltpu.sync_copy(idxp_h.at[s * NSUPS], supidx)
        build_and_issue(0, 0)

        def pair(kk, carry):
            i0 = 2 * kk
            subiter(i0, 0, True)
            subiter(i0 + 1, 1, True)
            return carry

        lax.fori_loop(0, NPAIR, pair, 0)
        subiter(NCH - 1, (NCH - 1) % 2, False)

        wait_writes(0)
        wait_writes(1)

        plsc.subcore_barrier()

        @pl.when(s < NS - 1)
        def _():
            r0 = s * STRIPE
            pltpu.sync_copy(acc_s.at[pl.ds(r0, STRIPE)],
                            acc_h.at[pl.ds(c * N + r0, STRIPE)])

        @pl.when(s == NS - 1)
        def _():
            r0 = 15 * STRIPE
            pltpu.sync_copy(acc_s.at[pl.ds(r0, STRIPE_LAST)],
                            acc_h.at[pl.ds(c * N + r0, STRIPE_LAST)])

    if with_eij:
        @functools.partial(
            pl.kernel,
            out_type=out_type,
            mesh=mesh,
            compiler_params=pltpu.CompilerParams(use_tc_tiling_on_sc=False),
            scratch_types=scratch,
        )
        def k(idxp_h, uv_h, efp_h, tdb_h, te_h, acc_h, eij_h,
              supidx, sv0, sv1, de0, de1, dv0, dv1,
              db0, db1, eh0, eh1, cv0, cv1, ct0, ct1,
              six0, six1, acc_s,
              eo0, eo1, uvec, vvec, supef,
              ga0, gb0_, ga1, gb1_, cs0, cs1, ws0, ws1):
            _body(idxp_h, None, uv_h, efp_h, tdb_h, te_h, acc_h, eij_h,
                  supidx, (sv0, sv1), (de0, de1), (dv0, dv1),
                  (db0, db1), (eh0, eh1), (cv0, cv1), (ct0, ct1),
                  (six0, six1), acc_s, (eo0, eo1), uvec, vvec, supef,
                  ((ga0, gb0_), (ga1, gb1_)), None, (cs0, cs1),
                  (ws0, ws1))
    else:
        @functools.partial(
            pl.kernel,
            out_type=out_type,
            mesh=mesh,
            compiler_params=pltpu.CompilerParams(use_tc_tiling_on_sc=False),
            scratch_types=scratch,
        )
        def k(idxp_h, ce_h, tdb_h, te_h, acc_h,
              supidx, sv0, sv1, de0, de1, dv0, dv1,
              db0, db1, eh0, eh1, cv0, cv1, ct0, ct1,
              six0, six1, acc_s,
              ga0, gb0_, es0, ga1, gb1_, es1, cs0, cs1):
            _body(idxp_h, ce_h, None, None, tdb_h, te_h, acc_h, None,
                  supidx, (sv0, sv1), (de0, de1), (dv0, dv1),
                  (db0, db1), (eh0, eh1), (cv0, cv1), (ct0, ct1),
                  (six0, six1), acc_s, None, None, None, None,
                  ((ga0, gb0_), (ga1, gb1_)), (es0, es1), (cs0, cs1),
                  None)

    if with_eij:
        return k(idxp, ce_or_uv, efp, tdb, te)
    return k(idxp, ce_or_uv, tdb, te)


def kernel(edge_index, nodes_feat, edges_feat, nodes_num_norm_sqrt,
           edges_num_norm_sqrt, params):
    p = params
    src = edge_index[0].astype(jnp.int32)
    dst = edge_index[1].astype(jnp.int32)
    idxp = jnp.stack([src.reshape(-1, SUPE), dst.reshape(-1, SUPE)], axis=1)
    efp = edges_feat[:, 0].reshape(-1, SUPE)

    we2 = jnp.stack([p['emb_e_w'][:, 0], p['emb_e_b']], axis=0)
    h0, uv = _emb_call(
        nodes_feat, p['emb_h_w'], p['emb_h_b'][None, :], we2,
        p['l0_C_w'], p['l0_C_b'][None, :])
    def wpack(l):
        w4 = jnp.concatenate(
            [p[f'l{l}_A_w'], p[f'l{l}_B_w'], p[f'l{l}_D_w'], p[f'l{l}_E_w']],
            axis=0)
        b4 = jnp.stack(
            [p[f'l{l}_A_b'], p[f'l{l}_B_b'], p[f'l{l}_D_b'], p[f'l{l}_E_b']],
            axis=0)
        return w4, b4

    w4, b4 = wpack(0)
    ah0, db0, e0s = _proj_call(h0, w4, b4)
    acc0, eij0 = _sc_edge(
        idxp, uv,
        db0.reshape(NC * N, D), e0s.reshape(NC * N, H), True, efp)
    gb0 = jnp.stack([p['l0_bn_h_g'], p['l0_bn_h_b']], axis=0)
    h1 = _hupdate(acc0.reshape(NC, N, D), ah0, h0, nodes_num_norm_sqrt, gb0)

    w4, b4 = wpack(1)
    ah1, db1, e1s = _proj_call(h1, w4, b4)
    stats0 = _estats_call(eij0.reshape(NC, E, H), edges_num_norm_sqrt)
    scal = jnp.stack(
        [p['emb_e_w'][:, 0], p['emb_e_b'], p['l0_bn_e_g'], p['l0_bn_e_b'],
         p['l1_C_b']], axis=0)
    ce1 = _ce1_call(eij0.reshape(NC, E, H), edges_feat, edges_num_norm_sqrt,
                    stats0, scal, p['l1_C_w'])
    acc1, = _sc_edge(
        idxp, ce1.reshape(NC * E, H),
        db1.reshape(NC * N, D), e1s.reshape(NC * N, H), False)

    gb1 = jnp.stack([p['l1_bn_h_g'], p['l1_bn_h_b']], axis=0)
    logits = _final_call(acc1.reshape(NC, N, D), ah1, h1,
                         nodes_num_norm_sqrt, gb1, p['out_w'])
    return logits


# speedup vs baseline: 1.0493x; 1.0493x over previous
import functools

import jax
import jax.numpy as jnp
from jax import lax
from jax.experimental import pallas as pl
from jax.experimental.pallas import tpu as pltpu
from jax.experimental.pallas import tpu_sc as plsc

N = 10000
E = 320000
D = 128
H = 64
NC = 2
NS = 16
CHUNK = 32
E_PER_SUB = E // NS
SUPE = 800
SUPC = SUPE // CHUNK
NSUPS = E_PER_SUB // SUPE
NCH = E_PER_SUB // CHUNK
NPAIR = (NCH - 1) // 2
STRIPE = 624
STRIPE_LAST = N - 15 * STRIPE
F32 = jnp.float32


def _emb_body(nf_ref, wh_ref, bh_ref, we_ref, cw_ref, cb_ref, h_ref, uv_ref):
    x = nf_ref[...]
    h_ref[...] = jnp.dot(x, wh_ref[...].T, preferred_element_type=F32) + bh_ref[...]
    we = we_ref[...]
    cw = cw_ref[...]
    u = jnp.dot(we[0:1], cw.T, preferred_element_type=F32)
    v = jnp.dot(we[1:2], cw.T, preferred_element_type=F32) + cb_ref[...]
    uv_ref[...] = jnp.concatenate([u, v], axis=0)


def _emb_call(nodes_feat, wh, bh, we2, cw, cb):
    blk = 2000
    grid = N // blk
    return pl.pallas_call(
        _emb_body,
        grid=(grid,),
        in_specs=[
            pl.BlockSpec((blk, D), lambda i: (i, 0)),
            pl.BlockSpec((D, D), lambda i: (0, 0)),
            pl.BlockSpec((1, D), lambda i: (0, 0)),
            pl.BlockSpec((2, D), lambda i: (0, 0)),
            pl.BlockSpec((D, D), lambda i: (0, 0)),
            pl.BlockSpec((1, D), lambda i: (0, 0)),
        ],
        out_specs=[
            pl.BlockSpec((blk, D), lambda i: (i, 0)),
            pl.BlockSpec((2, D), lambda i: (0, 0)),
        ],
        out_shape=[
            jax.ShapeDtypeStruct((N, D), F32),
            jax.ShapeDtypeStruct((2, D), F32),
        ],
    )(nodes_feat, wh, bh, we2, cw, cb)


def _ce0_body(ef_ref, uv_ref, ce_ref):
    uv = uv_ref[...]
    ce = ef_ref[...] * uv[0:1] + uv[1:2]
    ce_ref[...] = jnp.stack([ce[:, :H], ce[:, H:]], axis=0)


def _ce0_call(ef, uv):
    blk = 2000
    grid = E // blk
    return pl.pallas_call(
        _ce0_body,
        grid=(grid,),
        in_specs=[
            pl.BlockSpec((blk, 1), lambda i: (i, 0)),
            pl.BlockSpec((2, D), lambda i: (0, 0)),
        ],
        out_specs=pl.BlockSpec((2, blk, H), lambda i: (0, i, 0)),
        out_shape=jax.ShapeDtypeStruct((2, E, H), F32),
    )(ef, uv)


def _proj_body(h_ref, w_ref, b_ref, a_ref, db_ref, es_ref):
    x = h_ref[...]
    w = w_ref[...]
    b = b_ref[...]
    a_ref[...] = jnp.dot(x, w[0:D].T, preferred_element_type=F32) + b[0]
    bh = jnp.dot(x, w[D:2 * D].T, preferred_element_type=F32) + b[1]
    dh = jnp.dot(x, w[2 * D:3 * D].T, preferred_element_type=F32) + b[2]
    eh = jnp.dot(x, w[3 * D:4 * D].T, preferred_element_type=F32) + b[3]
    db_ref[...] = jnp.stack(
        [jnp.concatenate([dh[:, :H], bh[:, :H]], axis=1),
         jnp.concatenate([dh[:, H:], bh[:, H:]], axis=1)], axis=0)
    es_ref[...] = jnp.stack([eh[:, :H], eh[:, H:]], axis=0)


def _proj_call(h, w4, b4):
    blk = 2000
    grid = N // blk
    return pl.pallas_call(
        _proj_body,
        grid=(grid,),
        in_specs=[
            pl.BlockSpec((blk, D), lambda i: (i, 0)),
            pl.BlockSpec((4 * D, D), lambda i: (0, 0)),
            pl.BlockSpec((4, D), lambda i: (0, 0)),
        ],
        out_specs=[
            pl.BlockSpec((blk, D), lambda i: (i, 0)),
            pl.BlockSpec((2, blk, D), lambda i: (0, i, 0)),
            pl.BlockSpec((2, blk, H), lambda i: (0, i, 0)),
        ],
        out_shape=[
            jax.ShapeDtypeStruct((N, D), F32),
            jax.ShapeDtypeStruct((2, N, D), F32),
            jax.ShapeDtypeStruct((2, N, H), F32),
        ],
    )(h, w4, b4)


def _estats_body(eij_ref, en_ref, st_ref):
    i = pl.program_id(0)

    @pl.when(i == 0)
    def _():
        st_ref[...] = jnp.zeros_like(st_ref)

    eij = eij_ref[...]
    x = jnp.concatenate([eij[0], eij[1]], axis=1) * en_ref[...]
    st_ref[0:1, :] += jnp.sum(x, axis=0, keepdims=True)
    st_ref[1:2, :] += jnp.sum(jnp.square(x), axis=0, keepdims=True)


def _estats_call(eij, en):
    blk = 4000
    grid = E // blk
    return pl.pallas_call(
        _estats_body,
        grid=(grid,),
        in_specs=[
            pl.BlockSpec((2, blk, H), lambda i: (0, i, 0)),
            pl.BlockSpec((blk, 1), lambda i: (i, 0)),
        ],
        out_specs=pl.BlockSpec((8, D), lambda i: (0, 0)),
        out_shape=jax.ShapeDtypeStruct((8, D), F32),
    )(eij, en)


def _hupd_body(acc_ref, ah_ref, hin_ref, nn_ref, gb_ref, out_ref):
    acc = acc_ref[...]
    num = jnp.concatenate([acc[0, :, :H], acc[1, :, :H]], axis=1)
    den = jnp.concatenate([acc[0, :, H:], acc[1, :, H:]], axis=1)
    x = (ah_ref[...] + num / (den + 1e-6)) * nn_ref[...]
    mu = jnp.mean(x, axis=0, keepdims=True)
    var = jnp.mean(jnp.square(x), axis=0, keepdims=True) - jnp.square(mu)
    gb = gb_ref[...]
    x = (x - mu) / jnp.sqrt(var + 1e-5) * gb[0:1] + gb[1:2]
    out_ref[...] = hin_ref[...] + jnp.maximum(x, 0.0)


def _hupdate(acc, ah, h_in, nn, gb):
    return pl.pallas_call(
        _hupd_body,
        out_shape=jax.ShapeDtypeStruct((N, D), F32),
    )(acc, ah, h_in, nn, gb)


def _final_body(acc_ref, ah_ref, hin_ref, nn_ref, gb_ref, ow_ref, out_ref):
    acc = acc_ref[...]
    num = jnp.concatenate([acc[0, :, :H], acc[1, :, :H]], axis=1)
    den = jnp.concatenate([acc[0, :, H:], acc[1, :, H:]], axis=1)
    x = (ah_ref[...] + num / (den + 1e-6)) * nn_ref[...]
    mu = jnp.mean(x, axis=0, keepdims=True)
    var = jnp.mean(jnp.square(x), axis=0, keepdims=True) - jnp.square(mu)
    gb = gb_ref[...]
    x = (x - mu) / jnp.sqrt(var + 1e-5) * gb[0:1] + gb[1:2]
    h = hin_ref[...] + jnp.maximum(x, 0.0)
    hg = jnp.mean(h, axis=0, keepdims=True)
    out_ref[...] = jnp.dot(hg, ow_ref[...].T, preferred_element_type=F32)


def _final_call(acc, ah, h_in, nn, gb, out_w):
    n_classes = out_w.shape[0]
    return pl.pallas_call(
        _final_body,
        out_shape=jax.ShapeDtypeStruct((1, n_classes), F32),
    )(acc, ah, h_in, nn, gb, out_w)


def _ce1_body(eij_ref, ef_ref, en_ref, st_ref, sc_ref, cw_ref, ce_ref):
    st = st_ref[...]
    mean = st[0] / float(E)
    var = st[1] / float(E) - jnp.square(mean)
    inv = 1.0 / jnp.sqrt(var + 1e-5)

    sc = sc_ref[...]
    eij = eij_ref[...]
    x = jnp.concatenate([eij[0], eij[1]], axis=1) * en_ref[...]
    x = (x - mean[None, :]) * inv[None, :] * sc[2:3] + sc[3:4]
    e0 = ef_ref[...] * sc[0:1] + sc[1:2]
    el1 = e0 + jnp.maximum(x, 0.0)
    ce = jnp.dot(el1, cw_ref[...].T, preferred_element_type=F32) + sc[4:5]
    ce_ref[...] = jnp.stack([ce[:, :H], ce[:, H:]], axis=0)


def _ce1_call(eij, ef, en, stats, scal, cw):
    blk = 2000
    grid = E // blk
    return pl.pallas_call(
        _ce1_body,
        grid=(grid,),
        in_specs=[
            pl.BlockSpec((2, blk, H), lambda i: (0, i, 0)),
            pl.BlockSpec((blk, 1), lambda i: (i, 0)),
            pl.BlockSpec((blk, 1), lambda i: (i, 0)),
            pl.BlockSpec((8, D), lambda i: (0, 0)),
            pl.BlockSpec((5, D), lambda i: (0, 0)),
            pl.BlockSpec((D, D), lambda i: (0, 0)),
        ],
        out_specs=pl.BlockSpec((2, blk, H), lambda i: (0, i, 0)),
        out_shape=jax.ShapeDtypeStruct((2, E, H), F32),
    )(eij, ef, en, stats, scal, cw)


def _sigmoid(x):
    return 1.0 / (1.0 + jnp.exp(-x))


def _sc_edge(idxp, ce, tdb, te, with_eij):
    mesh = plsc.VectorSubcoreMesh(
        core_axis_name="c", subcore_axis_name="s", num_cores=NC,
        num_subcores=NS)

    out_type = [jax.ShapeDtypeStruct((NC * N, D), F32)]
    if with_eij:
        out_type.append(jax.ShapeDtypeStruct((NC * E, H), F32))

    scratch = [
        pltpu.VMEM((2, SUPE), jnp.int32),
        pltpu.VMEM((CHUNK,), jnp.int32),
        pltpu.VMEM((CHUNK,), jnp.int32),
        pltpu.VMEM((CHUNK,), jnp.int32),
        pltpu.VMEM((CHUNK,), jnp.int32),
        pltpu.VMEM((CHUNK,), jnp.int32),
        pltpu.VMEM((CHUNK,), jnp.int32),
        pltpu.VMEM((CHUNK, D), F32),
        pltpu.VMEM((CHUNK, D), F32),
        pltpu.VMEM((CHUNK, H), F32),
        pltpu.VMEM((CHUNK, H), F32),
        pltpu.VMEM((CHUNK, H), F32),
        pltpu.VMEM((CHUNK, H), F32),
        pltpu.VMEM((CHUNK, D), F32),
        pltpu.VMEM((CHUNK, D), F32),
        pltpu.VMEM((CHUNK,), jnp.int32),
        pltpu.VMEM((CHUNK,), jnp.int32),
        pltpu.VMEM_SHARED((N, D), F32),
    ]
    if with_eij:
        scratch += [
            pltpu.VMEM((CHUNK, H), F32),
            pltpu.VMEM((CHUNK, H), F32),
        ]
    nsem = 10 if with_eij else 8
    scratch += [pltpu.SemaphoreType.DMA] * nsem

    def _body(idxp_h, ce_h, tdb_h, te_h, acc_h, eij_h,
              supidx, SV, DE, DV, DB, EH, CV, CT, SIX, acc_s, EO,
              GS, ES, CS, WS):
        c = lax.axis_index("c")
        s = lax.axis_index("s")
        cN = c * N

        ct0 = CT[0]

        def zrow(i, _):
            for f in range(D // 16):
                ct0[i, pl.ds(f * 16, 16)] = jnp.zeros((16,), F32)
            return 0
        lax.fori_loop(0, 16, zrow, 0)

        nz = lax.select(s == NS - 1, STRIPE_LAST // 16, STRIPE // 16)

        def zcopy(kk, _):
            pltpu.sync_copy(ct0.at[pl.ds(0, 16)],
                            acc_s.at[pl.ds(s * STRIPE + kk * 16, 16)])
            return 0
        lax.fori_loop(0, nz, zcopy, 0)

        plsc.subcore_barrier()

        def build_and_issue(ip1, q):
            off1 = (ip1 % SUPC) * CHUNK
            for g in range(CHUNK // 16):
                sl = pl.ds(off1 + g * 16, 16)
                dsl = pl.ds(g * 16, 16)
                srcs = supidx[0, sl]
                dsts = supidx[1, sl]
                SV[q][dsl] = srcs + cN
                DV[q][dsl] = dsts
                DE[q][dsl] = dsts + cN
            pltpu.async_copy(tdb_h.at[SV[q]], DB[q], GS[q][0])
            pltpu.async_copy(te_h.at[DE[q]], EH[q], GS[q][1])
            cbase1 = c * E + s * E_PER_SUB + ip1 * CHUNK
            pltpu.async_copy(ce_h.at[pl.ds(cbase1, CHUNK)], CV[q], ES[q])

        def wait_in(p):
            pltpu.make_async_copy(
                tdb_h.at[pl.ds(0, CHUNK)], DB[p], GS[p][0]).wait()
            pltpu.make_async_copy(
                te_h.at[pl.ds(0, CHUNK)], EH[p], GS[p][1]).wait()
            pltpu.make_async_copy(
                ce_h.at[pl.ds(0, CHUNK)], CV[p], ES[p]).wait()

        def compute(i, p):
            db, eh, cv, ct = DB[p], EH[p], CV[p], CT[p]
            eo = EO[p] if with_eij else None

            def grp(g, carry):
                for j in range(4):
                    r = g * 4 + j
                    for f in range(H // 16):
                        sl = pl.ds(f * 16, 16)
                        sl2 = pl.ds(H + f * 16, 16)
                        eij = db[r, sl] + eh[r, sl] + cv[r, sl]
                        sig = _sigmoid(eij)
                        if with_eij:
                            eo[r, sl] = eij
                        ct[r, sl] = sig * db[r, sl2]
                        ct[r, sl2] = sig
                return carry

            lax.fori_loop(0, CHUNK // 4, grp, 0)

        def issue_writes(i, p):
            for g in range(CHUNK // 16):
                dsl = pl.ds(g * 16, 16)
                SIX[p][dsl] = DV[p][dsl]
            pltpu.async_copy(CT[p], acc_s.at[SIX[p]], CS[p], add=True)
            if with_eij:
                cbase = c * E + s * E_PER_SUB + i * CHUNK
                pltpu.async_copy(EO[p], eij_h.at[pl.ds(cbase, CHUNK)], WS[p])

        def wait_writes(p):
            pltpu.make_async_copy(
                CT[p], acc_s.at[pl.ds(0, CHUNK)], CS[p]).wait()
            if with_eij:
                pltpu.make_async_copy(
                    EO[p], eij_h.at[pl.ds(0, CHUNK)], WS[p]).wait()

        def subiter(i, p, issue_next):
            if issue_next:
                ip1 = i + 1

                @pl.when((ip1 % SUPC) == 0)
                def _():
                    pltpu.sync_copy(idxp_h.at[s * NSUPS + ip1 // SUPC], supidx)

                build_and_issue(ip1, p ^ 1)

            wait_in(p)

            @pl.when(i >= 2)
            def _():
                wait_writes(p)

            compute(i, p)
            issue_writes(i, p)

        pltpu.sync_copy(idxp_h.at[s * NSUPS], supidx)
        build_and_issue(0, 0)

        def pair(kk, carry):
            i0 = 2 * kk
            subiter(i0, 0, True)
            subiter(i0 + 1, 1, True)
            return carry

        lax.fori_loop(0, NPAIR, pair, 0)
        subiter(NCH - 1, (NCH - 1) % 2, False)

        wait_writes(0)
        wait_writes(1)

        plsc.subcore_barrier()

        @pl.when(s < NS - 1)
        def _():
            r0 = s * STRIPE
            pltpu.sync_copy(acc_s.at[pl.ds(r0, STRIPE)],
                            acc_h.at[pl.ds(c * N + r0, STRIPE)])

        @pl.when(s == NS - 1)
        def _():
            r0 = 15 * STRIPE
            pltpu.sync_copy(acc_s.at[pl.ds(r0, STRIPE_LAST)],
                            acc_h.at[pl.ds(c * N + r0, STRIPE_LAST)])

    if with_eij:
        @functools.partial(
            pl.kernel,
            out_type=out_type,
            mesh=mesh,
            compiler_params=pltpu.CompilerParams(use_tc_tiling_on_sc=False),
            scratch_types=scratch,
        )
        def k(idxp_h, ce_h, tdb_h, te_h, acc_h, eij_h,
              supidx, sv0, sv1, de0, de1, dv0, dv1,
              db0, db1, eh0, eh1, cv0, cv1, ct0, ct1, six0, six1, acc_s,
              eo0, eo1,
              ga0, gb0_, es0, ga1, gb1_, es1, cs0, cs1, ws0, ws1):
            _body(idxp_h, ce_h, tdb_h, te_h, acc_h, eij_h,
                  supidx, (sv0, sv1), (de0, de1), (dv0, dv1),
                  (db0, db1), (eh0, eh1), (cv0, cv1), (ct0, ct1),
                  (six0, six1), acc_s, (eo0, eo1),
                  ((ga0, gb0_), (ga1, gb1_)), (es0, es1), (cs0, cs1),
                  (ws0, ws1))
    else:
        @functools.partial(
            pl.kernel,
            out_type=out_type,
            mesh=mesh,
            compiler_params=pltpu.CompilerParams(use_tc_tiling_on_sc=False),
            scratch_types=scratch,
        )
        def k(idxp_h, ce_h, tdb_h, te_h, acc_h,
              supidx, sv0, sv1, de0, de1, dv0, dv1,
              db0, db1, eh0, eh1, cv0, cv1, ct0, ct1, six0, six1, acc_s,
              ga0, gb0_, es0, ga1, gb1_, es1, cs0, cs1):
            _body(idxp_h, ce_h, tdb_h, te_h, acc_h, None,
                  supidx, (sv0, sv1), (de0, de1), (dv0, dv1),
                  (db0, db1), (eh0, eh1), (cv0, cv1), (ct0, ct1),
                  (six0, six1), acc_s, None,
                  ((ga0, gb0_), (ga1, gb1_)), (es0, es1), (cs0, cs1),
                  None)

    return k(idxp, ce, tdb, te)


def kernel(edge_index, nodes_feat, edges_feat, nodes_num_norm_sqrt,
           edges_num_norm_sqrt, params):
    p = params
    src = edge_index[0].astype(jnp.int32)
    dst = edge_index[1].astype(jnp.int32)
    idxp = jnp.stack([src.reshape(-1, SUPE), dst.reshape(-1, SUPE)], axis=1)

    we2 = jnp.stack([p['emb_e_w'][:, 0], p['emb_e_b']], axis=0)
    h0, uv = _emb_call(
        nodes_feat, p['emb_h_w'], p['emb_h_b'][None, :], we2,
        p['l0_C_w'], p['l0_C_b'][None, :])
    ce0 = _ce0_call(edges_feat, uv)

    def wpack(l):
        w4 = jnp.concatenate(
            [p[f'l{l}_A_w'], p[f'l{l}_B_w'], p[f'l{l}_D_w'], p[f'l{l}_E_w']],
            axis=0)
        b4 = jnp.stack(
            [p[f'l{l}_A_b'], p[f'l{l}_B_b'], p[f'l{l}_D_b'], p[f'l{l}_E_b']],
            axis=0)
        return w4, b4

    w4, b4 = wpack(0)
    ah0, db0, e0s = _proj_call(h0, w4, b4)
    acc0, eij0 = _sc_edge(
        idxp, ce0.reshape(NC * E, H),
        db0.reshape(NC * N, D), e0s.reshape(NC * N, H), True)
    gb0 = jnp.stack([p['l0_bn_h_g'], p['l0_bn_h_b']], axis=0)
    h1 = _hupdate(acc0.reshape(NC, N, D), ah0, h0, nodes_num_norm_sqrt, gb0)

    w4, b4 = wpack(1)
    ah1, db1, e1s = _proj_call(h1, w4, b4)
    stats0 = _estats_call(eij0.reshape(NC, E, H), edges_num_norm_sqrt)
    scal = jnp.stack(
        [p['emb_e_w'][:, 0], p['emb_e_b'], p['l0_bn_e_g'], p['l0_bn_e_b'],
         p['l1_C_b']], axis=0)
    ce1 = _ce1_call(eij0.reshape(NC, E, H), edges_feat, edges_num_norm_sqrt,
                    stats0, scal, p['l1_C_w'])
    acc1, = _sc_edge(
        idxp, ce1.reshape(NC * E, H),
        db1.reshape(NC * N, D), e1s.reshape(NC * N, H), False)

    gb1 = jnp.stack([p['l1_bn_h_g'], p['l1_bn_h_b']], axis=0)
    logits = _final_call(acc1.reshape(NC, N, D), ah1, h1,
                         nodes_num_norm_sqrt, gb1, p['out_w'])
    return logits
